# Initial kernel scaffold; baseline (speedup 1.0000x reference)
import jax
import jax.numpy as jnp
from jax.experimental import pallas as pl


def kernel(x, W1, b1, W2, b2, Wr, br):
    raise NotImplementedError("write your pallas kernel here")


# speedup vs baseline: 1.4274x; 1.4274x over previous
import functools

import jax
import jax.numpy as jnp
from jax import lax
from jax.experimental import pallas as pl
from jax.experimental.pallas import tpu as pltpu
from jax.experimental.pallas import tpu_sc as plsc

B, S, D_MODEL, D_FF, E = 1, 2048, 1024, 4096, 8
TB = 128
NBP = S // TB + (E - 1)
NROWS = NBP * TB
FFT = 1024
NFF = D_FF // FFT
NMETA = 32


def _plan_body(x_ref, wr_ref, br_ref, dest_ref, meta_ref):
    logits = jnp.dot(x_ref[...], wr_ref[...],
                     preferred_element_type=jnp.float32) + br_ref[...]
    m = jnp.max(logits, axis=1, keepdims=True)
    e_iota = lax.broadcasted_iota(jnp.int32, (S, E), 1)
    idx = jnp.min(jnp.where(logits >= m, e_iota, E), axis=1, keepdims=True)
    oh = (e_iota == idx).astype(jnp.float32)

    counts = jnp.sum(oh, axis=0, keepdims=True).astype(jnp.int32)
    cpad = ((counts + TB - 1) // TB) * TB
    er = lax.broadcasted_iota(jnp.int32, (E, E), 0)
    ec = lax.broadcasted_iota(jnp.int32, (E, E), 1)
    offs = jnp.dot(cpad.astype(jnp.float32), (er < ec).astype(jnp.float32),
                   preferred_element_type=jnp.float32).astype(jnp.int32)

    C = 512
    G = S // C
    oh3 = oh.reshape(G, C, E)
    csum = jnp.sum(oh3, axis=1)
    gr = lax.broadcasted_iota(jnp.int32, (G, G), 0)
    gc = lax.broadcasted_iota(jnp.int32, (G, G), 1)
    base = jnp.dot((gr > gc).astype(jnp.float32), csum,
                   preferred_element_type=jnp.float32)
    rr = lax.broadcasted_iota(jnp.int32, (C, C), 0)
    rc = lax.broadcasted_iota(jnp.int32, (C, C), 1)
    ltri = (rr > rc).astype(jnp.float32)
    ranks = [jnp.dot(ltri, oh3[g], preferred_element_type=jnp.float32)
             + base[g][None, :] for g in range(G)]
    rank_full = jnp.concatenate(ranks, axis=0)
    rank = jnp.sum(rank_full * oh, axis=1, keepdims=True)

    dest_off = jnp.sum(offs.astype(jnp.float32) * oh, axis=1, keepdims=True)
    dest_ref[...] = (dest_off + rank).astype(jnp.int32)

    nblk = jnp.sum(cpad, axis=1, keepdims=True) // TB
    blk_start = offs // TB
    b_iota = lax.broadcasted_iota(jnp.int32, (NMETA, 1), 0)
    b_cl = jnp.minimum(b_iota, nblk - 1)
    blk_e = jnp.sum((b_cl >= blk_start).astype(jnp.int32),
                    axis=1, keepdims=True) - 1
    valid = (b_iota < nblk).astype(jnp.int32)
    pad = jnp.zeros((NMETA, E - 3), jnp.int32)
    meta_ref[...] = jnp.concatenate([blk_e, b_cl, valid, pad], axis=1)


def _plan(xf, Wr, br2):
    return pl.pallas_call(
        _plan_body,
        out_shape=(jax.ShapeDtypeStruct((S, 1), jnp.int32),
                   jax.ShapeDtypeStruct((NMETA, E), jnp.int32)),
    )(xf, Wr, br2)


def _sc_mesh():
    return plsc.VectorSubcoreMesh(core_axis_name="c", subcore_axis_name="s")


def _dispatch(xf, dest):
    info = plsc.get_sparse_core_info()
    nw = info.num_cores * info.num_subcores
    rows_w = S // nw

    @functools.partial(
        pl.kernel, mesh=_sc_mesh(),
        out_type=jax.ShapeDtypeStruct((NROWS, D_MODEL), jnp.float32),
        scratch_types=[pltpu.VMEM((rows_w,), jnp.int32),
                       pltpu.VMEM((rows_w, D_MODEL), jnp.float32),
                       pltpu.SemaphoreType.DMA],
    )
    def k(x_hbm, d_hbm, out_hbm, idx_v, rows_v, sem):
        wid = lax.axis_index("s") * info.num_cores + lax.axis_index("c")
        base = wid * rows_w
        pltpu.sync_copy(d_hbm.at[pl.ds(base, rows_w)], idx_v)
        pltpu.sync_copy(x_hbm.at[pl.ds(base, rows_w)], rows_v)
        pltpu.async_copy(rows_v, out_hbm.at[idx_v], sem).wait()

    return k(xf, dest)


def _combine(sorted_out, dest):
    info = plsc.get_sparse_core_info()
    nw = info.num_cores * info.num_subcores
    rows_w = S // nw

    @functools.partial(
        pl.kernel, mesh=_sc_mesh(),
        out_type=jax.ShapeDtypeStruct((S, D_MODEL), jnp.float32),
        scratch_types=[pltpu.VMEM((rows_w,), jnp.int32),
                       pltpu.VMEM((rows_w, D_MODEL), jnp.float32),
                       pltpu.SemaphoreType.DMA],
    )
    def k(s_hbm, d_hbm, out_hbm, idx_v, rows_v, sem):
        wid = lax.axis_index("s") * info.num_cores + lax.axis_index("c")
        base = wid * rows_w
        pltpu.sync_copy(d_hbm.at[pl.ds(base, rows_w)], idx_v)
        pltpu.async_copy(s_hbm.at[idx_v], rows_v, sem).wait()
        pltpu.sync_copy(rows_v, out_hbm.at[pl.ds(base, rows_w)])

    return k(sorted_out, dest)


def _ffn_body(e_ref, xb_ref, v_ref, xs_ref, w1_ref, b1_ref, w2_ref, b2_ref,
              out_ref, acc_ref):
    j = pl.program_id(1)
    b = pl.program_id(0)

    @pl.when(v_ref[b] > 0)
    def _():
        h = jnp.maximum(
            jnp.dot(xs_ref[...], w1_ref[0],
                    preferred_element_type=jnp.float32) + b1_ref[0, 0], 0.0)
        p = jnp.dot(h, w2_ref[0], preferred_element_type=jnp.float32)

        @pl.when(j == 0)
        def _():
            acc_ref[...] = p

        @pl.when(j > 0)
        def _():
            acc_ref[...] += p

        @pl.when(j == NFF - 1)
        def _():
            out_ref[...] = acc_ref[...] + b2_ref[0]


def _ffn(xs, W1, b1, W2, b2, blk_e, blk_xb, blk_v):
    def jeff(b, j, e_ref, xb_ref, v_ref):
        return jnp.where(v_ref[b] > 0, j, NFF - 1)

    grid_spec = pltpu.PrefetchScalarGridSpec(
        num_scalar_prefetch=3,
        grid=(NBP, NFF),
        in_specs=[
            pl.BlockSpec((TB, D_MODEL),
                         lambda b, j, e, xb, v: (xb[b], 0)),
            pl.BlockSpec((1, D_MODEL, FFT),
                         lambda b, j, e, xb, v: (e[b], 0, jeff(b, j, e, xb, v))),
            pl.BlockSpec((1, 1, 1, FFT),
                         lambda b, j, e, xb, v: (e[b], jeff(b, j, e, xb, v),
                                                 0, 0)),
            pl.BlockSpec((1, FFT, D_MODEL),
                         lambda b, j, e, xb, v: (e[b], jeff(b, j, e, xb, v), 0)),
            pl.BlockSpec((1, 1, D_MODEL),
                         lambda b, j, e, xb, v: (e[b], 0, 0)),
        ],
        out_specs=pl.BlockSpec((TB, D_MODEL),
                               lambda b, j, e, xb, v: (xb[b], 0)),
        scratch_shapes=[pltpu.VMEM((TB, D_MODEL), jnp.float32)],
    )
    return pl.pallas_call(
        _ffn_body,
        grid_spec=grid_spec,
        out_shape=jax.ShapeDtypeStruct((NROWS, D_MODEL), jnp.float32),
        compiler_params=pltpu.CompilerParams(
            dimension_semantics=("arbitrary", "arbitrary")),
    )(blk_e, blk_xb, blk_v, xs, W1,
      b1.reshape(E, NFF, 1, FFT), W2, b2.reshape(E, 1, D_MODEL))


def kernel(x, W1, b1, W2, b2, Wr, br):
    xf = x.reshape(S, D_MODEL)
    dest2d, meta = _plan(xf, Wr, br.reshape(1, E))
    dest = dest2d.reshape(S)
    xs = _dispatch(xf, dest)
    outs = _ffn(xs, W1, b1, W2, b2, meta[:NBP, 0], meta[:NBP, 1],
                meta[:NBP, 2])
    out = _combine(outs, dest)
    return out.reshape(B, S, D_MODEL)


# speedup vs baseline: 1.6114x; 1.1289x over previous
import functools

import jax
import jax.numpy as jnp
from jax import lax
from jax.experimental import pallas as pl
from jax.experimental.pallas import tpu as pltpu
from jax.experimental.pallas import tpu_sc as plsc

B, S, D_MODEL, D_FF, E = 1, 2048, 1024, 4096, 8
TB = 128
NBP = S // TB + (E - 1)
NROWS = NBP * TB
FFT = 1024
NFF = D_FF // FFT
NMETA = 32


def _plan_body(x_ref, wr_ref, br_ref, dest_ref, meta_ref):
    logits = jnp.dot(x_ref[...], wr_ref[...],
                     preferred_element_type=jnp.float32) + br_ref[...]
    m = jnp.max(logits, axis=1, keepdims=True)
    e_iota = lax.broadcasted_iota(jnp.int32, (S, E), 1)
    idx = jnp.min(jnp.where(logits >= m, e_iota, E), axis=1, keepdims=True)
    oh = (e_iota == idx).astype(jnp.float32)

    counts = jnp.sum(oh, axis=0, keepdims=True).astype(jnp.int32)
    cpad = ((counts + TB - 1) // TB) * TB
    er = lax.broadcasted_iota(jnp.int32, (E, E), 0)
    ec = lax.broadcasted_iota(jnp.int32, (E, E), 1)
    offs = jnp.dot(cpad.astype(jnp.float32), (er < ec).astype(jnp.float32),
                   preferred_element_type=jnp.float32).astype(jnp.int32)

    C = 512
    G = S // C
    oh3 = oh.reshape(G, C, E)
    csum = jnp.sum(oh3, axis=1)
    gr = lax.broadcasted_iota(jnp.int32, (G, G), 0)
    gc = lax.broadcasted_iota(jnp.int32, (G, G), 1)
    base = jnp.dot((gr > gc).astype(jnp.float32), csum,
                   preferred_element_type=jnp.float32)
    rr = lax.broadcasted_iota(jnp.int32, (C, C), 0)
    rc = lax.broadcasted_iota(jnp.int32, (C, C), 1)
    ltri = (rr > rc).astype(jnp.float32)
    ranks = [jnp.dot(ltri, oh3[g], preferred_element_type=jnp.float32)
             + base[g][None, :] for g in range(G)]
    rank_full = jnp.concatenate(ranks, axis=0)
    rank = jnp.sum(rank_full * oh, axis=1, keepdims=True)

    dest_off = jnp.sum(offs.astype(jnp.float32) * oh, axis=1, keepdims=True)
    dest_ref[...] = (dest_off + rank).astype(jnp.int32)

    nblk = jnp.sum(cpad, axis=1, keepdims=True) // TB
    blk_start = offs // TB
    b_iota = lax.broadcasted_iota(jnp.int32, (NMETA, 1), 0)
    b_cl = jnp.minimum(b_iota, nblk - 1)
    blk_e = jnp.sum((b_cl >= blk_start).astype(jnp.int32),
                    axis=1, keepdims=True) - 1
    valid = (b_iota < nblk).astype(jnp.int32)
    pad = jnp.zeros((NMETA, E - 3), jnp.int32)
    meta_ref[...] = jnp.concatenate([blk_e, b_cl, valid, pad], axis=1)


def _plan(xf, Wr, br2):
    return pl.pallas_call(
        _plan_body,
        out_shape=(jax.ShapeDtypeStruct((S, 1), jnp.int32),
                   jax.ShapeDtypeStruct((NMETA, E), jnp.int32)),
    )(xf, Wr, br2)


def _sc_mesh():
    return plsc.VectorSubcoreMesh(core_axis_name="c", subcore_axis_name="s")


def _dispatch(xf, dest):
    info = plsc.get_sparse_core_info()
    nw = info.num_cores * info.num_subcores
    rows_w = S // nw

    @functools.partial(
        pl.kernel, mesh=_sc_mesh(),
        out_type=jax.ShapeDtypeStruct((NROWS, D_MODEL), jnp.float32),
        scratch_types=[pltpu.VMEM((rows_w,), jnp.int32),
                       pltpu.VMEM((rows_w, D_MODEL), jnp.float32),
                       pltpu.SemaphoreType.DMA],
    )
    def k(x_hbm, d_hbm, out_hbm, idx_v, rows_v, sem):
        wid = lax.axis_index("s") * info.num_cores + lax.axis_index("c")
        base = wid * rows_w
        pltpu.sync_copy(d_hbm.at[pl.ds(base, rows_w)], idx_v)
        pltpu.sync_copy(x_hbm.at[pl.ds(base, rows_w)], rows_v)
        pltpu.async_copy(rows_v, out_hbm.at[idx_v], sem).wait()

    return k(xf, dest)


def _combine(sorted_out, dest):
    info = plsc.get_sparse_core_info()
    nw = info.num_cores * info.num_subcores
    rows_w = S // nw

    @functools.partial(
        pl.kernel, mesh=_sc_mesh(),
        out_type=jax.ShapeDtypeStruct((S, D_MODEL), jnp.float32),
        scratch_types=[pltpu.VMEM((rows_w,), jnp.int32),
                       pltpu.VMEM((rows_w, D_MODEL), jnp.float32),
                       pltpu.SemaphoreType.DMA],
    )
    def k(s_hbm, d_hbm, out_hbm, idx_v, rows_v, sem):
        wid = lax.axis_index("s") * info.num_cores + lax.axis_index("c")
        base = wid * rows_w
        pltpu.sync_copy(d_hbm.at[pl.ds(base, rows_w)], idx_v)
        pltpu.async_copy(s_hbm.at[idx_v], rows_v, sem).wait()
        pltpu.sync_copy(rows_v, out_hbm.at[pl.ds(base, rows_w)])

    return k(sorted_out, dest)


def _ffn_body(e_ref, xb_ref, v_ref, xs_ref, w1_ref, b1_ref, w2_ref, b2_ref,
              out_ref, acc_ref):
    j = pl.program_id(0)
    b = pl.program_id(1)

    @pl.when(v_ref[b] > 0)
    def _():
        h = jnp.maximum(
            jnp.dot(xs_ref[...], w1_ref[0],
                    preferred_element_type=jnp.float32) + b1_ref[0, 0], 0.0)
        p = jnp.dot(h, w2_ref[0], preferred_element_type=jnp.float32)
        row = xb_ref[b] * TB

        @pl.when(j == 0)
        def _():
            acc_ref[pl.ds(row, TB), :] = p

        @pl.when(j > 0)
        def _():
            acc_ref[pl.ds(row, TB), :] += p

        @pl.when(j == NFF - 1)
        def _():
            out_ref[...] = acc_ref[pl.ds(row, TB), :] + b2_ref[0]


def _ffn(xs, W1, b1, W2, b2, blk_e, blk_xb, blk_v):
    grid_spec = pltpu.PrefetchScalarGridSpec(
        num_scalar_prefetch=3,
        grid=(NFF, NBP),
        in_specs=[
            pl.BlockSpec((TB, D_MODEL),
                         lambda j, b, e, xb, v: (xb[b], 0)),
            pl.BlockSpec((1, D_MODEL, FFT),
                         lambda j, b, e, xb, v: (e[b], 0, j)),
            pl.BlockSpec((1, 1, 1, FFT),
                         lambda j, b, e, xb, v: (e[b], j, 0, 0)),
            pl.BlockSpec((1, FFT, D_MODEL),
                         lambda j, b, e, xb, v: (e[b], j, 0)),
            pl.BlockSpec((1, 1, D_MODEL),
                         lambda j, b, e, xb, v: (e[b], 0, 0)),
        ],
        out_specs=pl.BlockSpec(
            (TB, D_MODEL),
            lambda j, b, e, xb, v: (jnp.where(j == NFF - 1, xb[b], 0), 0)),
        scratch_shapes=[pltpu.VMEM((NROWS, D_MODEL), jnp.float32)],
    )
    return pl.pallas_call(
        _ffn_body,
        grid_spec=grid_spec,
        out_shape=jax.ShapeDtypeStruct((NROWS, D_MODEL), jnp.float32),
        compiler_params=pltpu.CompilerParams(
            dimension_semantics=("arbitrary", "arbitrary")),
    )(blk_e, blk_xb, blk_v, xs, W1,
      b1.reshape(E, NFF, 1, FFT), W2, b2.reshape(E, 1, D_MODEL))


def kernel(x, W1, b1, W2, b2, Wr, br):
    xf = x.reshape(S, D_MODEL)
    dest2d, meta = _plan(xf, Wr, br.reshape(1, E))
    dest = dest2d.reshape(S)
    xs = _dispatch(xf, dest)
    outs = _ffn(xs, W1, b1, W2, b2, meta[:NBP, 0], meta[:NBP, 1],
                meta[:NBP, 2])
    out = _combine(outs, dest)
    return out.reshape(B, S, D_MODEL)
